# baseline (speedup 1.0000x reference)
import functools

import jax
import jax.numpy as jnp
from jax import lax
from jax.experimental import pallas as pl
from jax.experimental.pallas import tpu as pltpu
from jax.experimental.pallas import tpu_sc as plsc

B = 16384
F = 128
NC = 2
NS = 16
NW = NC * NS
BPW = B // NW
C = 256
NCHUNK = BPW // C
L = 16


def _sc_body(uid_hbm, iid_hbm, p_hbm, q_hbm, bu_hbm, bi_hbm, out_hbm,
             idxu_v, idxi_v, pbuf, qbuf, bu_v, bi_v, out_v, sem):
    wid = lax.axis_index("s") * NC + lax.axis_index("c")
    base = wid * BPW
    pltpu.sync_copy(uid_hbm.at[pl.ds(base, BPW)], idxu_v)
    pltpu.sync_copy(iid_hbm.at[pl.ds(base, BPW)], idxi_v)
    for c in range(NCHUNK):
        c0 = c * C
        cp1 = pltpu.async_copy(p_hbm.at[idxu_v.at[pl.ds(c0, C)]], pbuf, sem)
        cp2 = pltpu.async_copy(q_hbm.at[idxi_v.at[pl.ds(c0, C)]], qbuf, sem)
        cp3 = pltpu.async_copy(bu_hbm.at[idxu_v.at[pl.ds(c0, C)]], bu_v, sem)
        cp4 = pltpu.async_copy(bi_hbm.at[idxi_v.at[pl.ds(c0, C)]], bi_v, sem)
        cp1.wait()
        cp2.wait()
        cp3.wait()
        cp4.wait()

        lanes = lax.iota(jnp.int32, L)

        def group(g, _):
            r0 = g * L
            ridx = r0 + lanes
            acc = None
            for f in range(F):
                cidx = jnp.full((L,), f, jnp.int32)
                prod = (plsc.load_gather(pbuf, [ridx, cidx])
                        * plsc.load_gather(qbuf, [ridx, cidx]))
                acc = prod if acc is None else acc + prod
            acc = acc + bu_v[pl.ds(r0, L)] + bi_v[pl.ds(r0, L)]
            out_v[pl.ds(c0 + r0, L)] = acc
            return 0

        lax.fori_loop(0, C // L, group, 0)
    pltpu.sync_copy(out_v, out_hbm.at[pl.ds(base, BPW)])


def kernel(user_id, item_id, P, Q, user_bias, item_bias):
    mesh = plsc.VectorSubcoreMesh(core_axis_name="c", subcore_axis_name="s",
                                  num_cores=NC, num_subcores=NS)
    run = functools.partial(
        pl.kernel,
        out_type=jax.ShapeDtypeStruct((B,), jnp.float32),
        mesh=mesh,
        compiler_params=pltpu.CompilerParams(needs_layout_passes=False),
        scratch_types=[
            pltpu.VMEM((BPW,), jnp.int32),
            pltpu.VMEM((BPW,), jnp.int32),
            pltpu.VMEM((C, F), jnp.float32),
            pltpu.VMEM((C, F), jnp.float32),
            pltpu.VMEM((C,), jnp.float32),
            pltpu.VMEM((C,), jnp.float32),
            pltpu.VMEM((BPW,), jnp.float32),
            pltpu.SemaphoreType.DMA,
        ],
    )(_sc_body)
    out = run(user_id.astype(jnp.int32), item_id.astype(jnp.int32),
              P, Q, user_bias.reshape(-1), item_bias.reshape(-1))
    return out.reshape(B, 1)
